# baseline (speedup 1.0000x reference)
import functools
import jax
import jax.numpy as jnp
from jax import lax
from jax.experimental import pallas as pl
from jax.experimental.pallas import tpu as pltpu
from jax.experimental.pallas import tpu_sc as plsc

B, F, D = 1024, 128, 1024
NW = 32
RPW = B // NW
DH = D // 2
NCH = DH // 16


def _sc_body(idx_hbm, key_hbm, out_hbm, idx_v, key_v, out_v):
    nc = 2
    wid = lax.axis_index("s") * nc + lax.axis_index("c")
    base = wid * RPW
    pltpu.sync_copy(idx_hbm.at[pl.ds(base, RPW)], idx_v)
    lane = lax.iota(jnp.int32, 16)

    def do_half(h):
        pltpu.sync_copy(key_hbm.at[:, pl.ds(h * DH, DH)], key_v)

        def b_loop(b, _):
            idx_rows = [idx_v[b, pl.ds(16 * g, 16)] for g in range(F // 16)]

            def c_loop(c, _):
                diota = (h * DH + c * 16) + lane
                acc = jnp.zeros((16,), jnp.float32)
                for f in range(F):
                    s = idx_rows[f // 16][f % 16]
                    spl = jnp.full((16,), s, jnp.int32)
                    kt = key_v[f, pl.ds(c * 16, 16)]
                    acc = acc + jnp.where(diota < spl, kt, -kt)
                res = jnp.where(acc > 0.0, 1.0, -1.0)
                out_v[b, pl.ds(h * DH + c * 16, 16)] = res
                return 0

            lax.fori_loop(0, NCH, c_loop, 0)
            return 0

        lax.fori_loop(0, RPW, b_loop, 0)

    do_half(0)
    do_half(1)
    pltpu.sync_copy(out_v, out_hbm.at[pl.ds(base, RPW)])


def sc_density(idx_i32, key):
    mesh = plsc.VectorSubcoreMesh(core_axis_name="c", subcore_axis_name="s")
    k = functools.partial(
        pl.kernel,
        mesh=mesh,
        out_type=jax.ShapeDtypeStruct((B, D), jnp.float32),
        scratch_types=[
            pltpu.VMEM((RPW, F), jnp.int32),
            pltpu.VMEM((F, DH), jnp.float32),
            pltpu.VMEM((RPW, D), jnp.float32),
        ],
    )(_sc_body)
    return k(idx_i32, key)


def kernel(x, key_weight, therm_weight):
    d_dim = key_weight.shape[1]
    idx = jnp.clip(jnp.round(x * float(d_dim)), 0.0, float(d_dim)).astype(jnp.int32)
    return sc_density(idx, key_weight)


# speedup vs baseline: 20.0675x; 20.0675x over previous
import jax
import jax.numpy as jnp
from jax.experimental import pallas as pl


def _density_block_kernel(x_ref, key_ref, out_ref):
    bblk, f_dim = x_ref.shape
    d_dim = key_ref.shape[1]
    x = x_ref[...]
    idx = jnp.clip(jnp.round(x * float(d_dim)), 0.0, float(d_dim)).astype(jnp.int32)
    idx16 = idx.astype(jnp.int16)
    diota = jax.lax.broadcasted_iota(jnp.int32, (1, d_dim), 1).astype(jnp.int16)
    key = key_ref[...]
    acc = jnp.zeros((bblk, d_dim), jnp.bfloat16)
    for f in range(f_dim):
        idxf = idx16[:, f : f + 1]
        kf = key[f : f + 1, :]
        acc = acc + jnp.where(diota < idxf, kf, -kf)
    accf = acc.astype(jnp.float32)
    out_ref[...] = jnp.where(accf > 0.0, 1.0, -1.0)


def kernel(x, key_weight, therm_weight):
    b, f_dim = x.shape
    d_dim = key_weight.shape[1]
    bblk = 128
    key_bf16 = key_weight.astype(jnp.bfloat16)
    return pl.pallas_call(
        _density_block_kernel,
        grid=(b // bblk,),
        in_specs=[
            pl.BlockSpec((bblk, f_dim), lambda i: (i, 0)),
            pl.BlockSpec((f_dim, d_dim), lambda i: (0, 0)),
        ],
        out_specs=pl.BlockSpec((bblk, d_dim), lambda i: (i, 0)),
        out_shape=jax.ShapeDtypeStruct((b, d_dim), jnp.float32),
    )(x, key_bf16)


# speedup vs baseline: 20.9896x; 1.0460x over previous
import jax
import jax.numpy as jnp
from jax.experimental import pallas as pl


def _density_block_kernel(x_ref, key_ref, out_ref):
    bblk, f_dim = x_ref.shape
    d_dim = key_ref.shape[1]
    x = x_ref[...]
    idx = jnp.clip(jnp.round(x * float(d_dim)), 0.0, float(d_dim)).astype(jnp.int32)
    idx16 = idx.astype(jnp.int16)
    diota = jax.lax.broadcasted_iota(jnp.int32, (1, d_dim), 1).astype(jnp.int16)
    key = key_ref[...]
    acc = jnp.zeros((bblk, d_dim), jnp.bfloat16)
    for f in range(f_dim):
        idxf = idx16[:, f : f + 1]
        kf = key[f : f + 1, :]
        acc = acc + jnp.where(diota < idxf, kf, -kf)
    accf = acc.astype(jnp.float32)
    out_ref[...] = jnp.where(accf > 0.0, 1.0, -1.0)


def kernel(x, key_weight, therm_weight):
    b, f_dim = x.shape
    d_dim = key_weight.shape[1]
    bblk = 256
    key_bf16 = key_weight.astype(jnp.bfloat16)
    return pl.pallas_call(
        _density_block_kernel,
        grid=(b // bblk,),
        in_specs=[
            pl.BlockSpec((bblk, f_dim), lambda i: (i, 0)),
            pl.BlockSpec((f_dim, d_dim), lambda i: (0, 0)),
        ],
        out_specs=pl.BlockSpec((bblk, d_dim), lambda i: (i, 0)),
        out_shape=jax.ShapeDtypeStruct((b, d_dim), jnp.float32),
    )(x, key_bf16)


# speedup vs baseline: 22.0863x; 1.0522x over previous
import jax
import jax.numpy as jnp
from jax.experimental import pallas as pl
from jax.experimental.pallas import tpu as pltpu


def _density_block_kernel(x_ref, key_ref, out_ref, kbf_ref):
    bblk, f_dim = x_ref.shape
    d_dim = key_ref.shape[1]

    @pl.when(pl.program_id(0) == 0)
    def _convert_key():
        kbf_ref[...] = key_ref[...].astype(jnp.bfloat16)

    x = x_ref[...]
    idx = jnp.clip(jnp.round(x * float(d_dim)), 0.0, float(d_dim)).astype(jnp.int32)
    idx16 = idx.astype(jnp.int16)
    diota = jax.lax.broadcasted_iota(jnp.int32, (1, d_dim), 1).astype(jnp.int16)
    key = kbf_ref[...]
    acc = jnp.zeros((bblk, d_dim), jnp.bfloat16)
    for f in range(f_dim):
        idxf = idx16[:, f : f + 1]
        kf = key[f : f + 1, :]
        acc = acc + jnp.where(diota < idxf, kf, -kf)
    accf = acc.astype(jnp.float32)
    out_ref[...] = jnp.where(accf > 0.0, 1.0, -1.0)


def kernel(x, key_weight, therm_weight):
    b, f_dim = x.shape
    d_dim = key_weight.shape[1]
    bblk = 256
    return pl.pallas_call(
        _density_block_kernel,
        grid=(b // bblk,),
        in_specs=[
            pl.BlockSpec((bblk, f_dim), lambda i: (i, 0)),
            pl.BlockSpec((f_dim, d_dim), lambda i: (0, 0)),
        ],
        out_specs=pl.BlockSpec((bblk, d_dim), lambda i: (i, 0)),
        out_shape=jax.ShapeDtypeStruct((b, d_dim), jnp.float32),
        scratch_shapes=[pltpu.VMEM((f_dim, d_dim), jnp.bfloat16)],
    )(x, key_weight)


# speedup vs baseline: 22.2068x; 1.0055x over previous
import jax
import jax.numpy as jnp
from jax.experimental import pallas as pl
from jax.experimental.pallas import tpu as pltpu


def _density_block_kernel(x_ref, key_ref, out_ref, kbf_ref):
    bblk, f_dim = x_ref.shape
    d_dim = key_ref.shape[1]

    @pl.when(pl.program_id(0) == 0)
    def _convert_key():
        kbf_ref[...] = key_ref[...].astype(jnp.bfloat16)

    x = x_ref[...]
    idx = jnp.clip(jnp.round(x * float(d_dim)), 0.0, float(d_dim)).astype(jnp.int32)
    idx16 = idx.astype(jnp.int16)
    diota = jax.lax.broadcasted_iota(jnp.int32, (1, d_dim), 1).astype(jnp.int16)
    key = kbf_ref[...]
    acc = jnp.zeros((bblk, d_dim), jnp.bfloat16)
    for f in range(f_dim):
        idxf = idx16[:, f : f + 1]
        kf = key[f : f + 1, :]
        acc = acc + jnp.where(diota < idxf, kf, -kf)
    accf = acc.astype(jnp.float32)
    out_ref[...] = jnp.where(accf > 0.0, 1.0, -1.0)


def kernel(x, key_weight, therm_weight):
    b, f_dim = x.shape
    d_dim = key_weight.shape[1]
    bblk = 512
    return pl.pallas_call(
        _density_block_kernel,
        grid=(b // bblk,),
        in_specs=[
            pl.BlockSpec((bblk, f_dim), lambda i: (i, 0)),
            pl.BlockSpec((f_dim, d_dim), lambda i: (0, 0)),
        ],
        out_specs=pl.BlockSpec((bblk, d_dim), lambda i: (i, 0)),
        out_shape=jax.ShapeDtypeStruct((b, d_dim), jnp.float32),
        scratch_shapes=[pltpu.VMEM((f_dim, d_dim), jnp.bfloat16)],
    )(x, key_weight)
